# baseline (speedup 1.0000x reference)
import functools

import jax
import jax.numpy as jnp
from jax import lax
from jax.experimental import pallas as pl
from jax.experimental.pallas import tpu as pltpu
from jax.experimental.pallas import tpu_sc as plsc

F = 8192
K = 512
TILE = 256

NS = 16
L = 16
CHUNK = F // NS
NV = CHUNK // L
SHIFTS = (26, 22, 18, 14, 10, 6, 2, 0)
WIDTHS = (1 << 30,) + tuple(1 << s for s in SHIFTS[:-1])
SH_PAD = 128


def _sc_prob(mask_hbm, out_hbm, vals_v, hist_v, stage_f, merge_i, merge_f,
             out_v, sh_hist, sh_f):
    cid = lax.axis_index("c")
    sid = lax.axis_index("s")
    base = sid * CHUNK

    pltpu.sync_copy(mask_hbm.at[pl.ds(base, CHUNK)], vals_v)

    iota = lax.iota(jnp.int32, L)
    ones_i = jnp.ones((L,), jnp.int32)

    lo = jnp.zeros((L,), jnp.int32)
    c_above = jnp.zeros((L,), jnp.int32)
    for r, sh in enumerate(SHIFTS):
        hist_v[...] = jnp.zeros((L,), jnp.int32)
        hi = lo + jnp.int32(WIDTHS[r])
        sh_vec = jnp.full((L,), sh, jnp.int32)

        def hbody(i, carry, lo=lo, hi=hi, sh_vec=sh_vec):
            v = vals_v[pl.ds(pl.multiple_of(i * L, L), L)]
            b = plsc.bitcast(v, jnp.int32)
            inr = (b >= lo) & (b < hi)
            bucket = jnp.clip(lax.shift_right_logical(b - lo, sh_vec), 0, 15)
            plsc.addupdate_scatter(hist_v, [bucket], ones_i, mask=inr)
            return carry

        lax.fori_loop(0, NV, hbody, jnp.int32(0))

        pltpu.sync_copy(hist_v, sh_hist.at[SH_PAD + r * NS + sid])
        plsc.subcore_barrier()
        pltpu.sync_copy(sh_hist.at[pl.ds(SH_PAD + r * NS, NS), :], merge_i)
        cnt = jnp.zeros((L,), jnp.int32)
        for w in range(NS):
            cnt = cnt + merge_i[w, :]
        suffix = lax.rev(plsc.cumsum(lax.rev(cnt, (0,))), (0,)) + c_above
        b_star = plsc.all_reduce_population_count(suffix >= K) - 1
        c_above = c_above + jnp.sum(jnp.where(iota > b_star, cnt, 0))
        lo = lo + b_star * jnp.int32(1 << sh)

    t = lo
    n_gt = c_above
    need = K - n_gt

    def abody(i, carry):
        n_eq, vmax = carry
        v = vals_v[pl.ds(pl.multiple_of(i * L, L), L)]
        b = plsc.bitcast(v, jnp.int32)
        n_eq = n_eq + plsc.all_reduce_population_count(b == t)
        return n_eq, jnp.maximum(vmax, v)

    n_eq, vmax = lax.fori_loop(
        0, NV, abody,
        (jnp.zeros((L,), jnp.int32), jnp.full((L,), -jnp.inf, jnp.float32)),
    )
    hist_v[...] = n_eq
    stage_f[...] = vmax
    pltpu.sync_copy(hist_v, sh_hist.at[SH_PAD + len(SHIFTS) * NS + sid])
    pltpu.sync_copy(stage_f, sh_f.at[SH_PAD + sid])
    plsc.subcore_barrier()
    pltpu.sync_copy(sh_hist.at[pl.ds(SH_PAD + len(SHIFTS) * NS, NS), :], merge_i)
    pltpu.sync_copy(sh_f.at[pl.ds(SH_PAD, NS), :], merge_f)
    eq_pref = jnp.zeros((L,), jnp.int32)
    vmax_all = jnp.full((L,), -jnp.inf, jnp.float32)
    for w in range(NS):
        eq_pref = eq_pref + jnp.where(jnp.int32(w) < sid, merge_i[w, :], 0)
        vmax_all = jnp.maximum(vmax_all, merge_f[w, :])
    gmax = jnp.max(vmax_all)

    def bbody(i, carry):
        run, esum = carry
        idx = pl.multiple_of(i * L, L)
        v = vals_v[pl.ds(idx, L)]
        b = plsc.bitcast(v, jnp.int32)
        gt = b > t
        eq = b == t
        eqi = eq.astype(jnp.int32)
        rank = eq_pref + run + (plsc.cumsum(eqi) - eqi)
        sel = gt | (eq & (rank < need))
        e = jnp.where(sel, jnp.exp(v - gmax), 0.0)
        out_v[pl.ds(idx, L)] = e
        return run + plsc.all_reduce_population_count(eq), esum + e

    _, esum = lax.fori_loop(
        0, NV, bbody,
        (jnp.zeros((L,), jnp.int32), jnp.zeros((L,), jnp.float32)),
    )
    stage_f[...] = jnp.sum(esum) + jnp.zeros((L,), jnp.float32)
    pltpu.sync_copy(stage_f, sh_f.at[SH_PAD + NS + sid])
    plsc.subcore_barrier()
    pltpu.sync_copy(sh_f.at[pl.ds(SH_PAD + NS, NS), :], merge_f)
    tot = jnp.zeros((L,), jnp.float32)
    for w in range(NS):
        tot = tot + merge_f[w, :]
    inv = 1.0 / tot

    def cbody(i, carry):
        idx = pl.multiple_of(i * L, L)
        out_v[pl.ds(idx, L)] = out_v[pl.ds(idx, L)] * inv
        return carry

    lax.fori_loop(0, NV, cbody, jnp.int32(0))

    @pl.when(cid == 0)
    def _write():
        pltpu.sync_copy(out_v, out_hbm.at[pl.ds(base, CHUNK)])


_sc_prob_call = functools.partial(
    pl.kernel,
    out_type=jax.ShapeDtypeStruct((F,), jnp.float32),
    mesh=plsc.VectorSubcoreMesh(core_axis_name="c", subcore_axis_name="s"),
    compiler_params=pltpu.CompilerParams(needs_layout_passes=False),
    scratch_types=[
        pltpu.VMEM((CHUNK,), jnp.float32),
        pltpu.VMEM((L,), jnp.int32),
        pltpu.VMEM((L,), jnp.float32),
        pltpu.VMEM((NS, L), jnp.int32),
        pltpu.VMEM((NS, L), jnp.float32),
        pltpu.VMEM((CHUNK,), jnp.float32),
        pltpu.VMEM_SHARED((SH_PAD + (len(SHIFTS) + 1) * NS, L), jnp.int32),
        pltpu.VMEM_SHARED((SH_PAD + 2 * NS, L), jnp.float32),
    ],
)(_sc_prob)


def _mul_kernel(prob_ref, x_ref, o_ref):
    o_ref[...] = x_ref[...] * prob_ref[...]


def kernel(inputs, mask):
    b = inputs.shape[0]
    prob = _sc_prob_call(mask.reshape(F)).reshape(1, F)
    return pl.pallas_call(
        _mul_kernel,
        grid=(b // TILE,),
        in_specs=[
            pl.BlockSpec((1, F), lambda i: (0, 0)),
            pl.BlockSpec((TILE, F), lambda i: (i, 0)),
        ],
        out_specs=pl.BlockSpec((TILE, F), lambda i: (i, 0)),
        out_shape=jax.ShapeDtypeStruct(inputs.shape, inputs.dtype),
    )(prob, inputs)


# speedup vs baseline: 1.2113x; 1.2113x over previous
import jax
import jax.numpy as jnp
from jax.experimental import pallas as pl
from jax.experimental.pallas import tpu as pltpu

F = 8192
K = 512
TILE = 256


def _fused(mask_ref, x_ref, o_ref, prob_ref):
    @pl.when(pl.program_id(0) == 0)
    def _compute_prob():
        m = mask_ref[...]
        bits = pltpu.bitcast(m, jnp.int32)

        def vbody(_, carry):
            lo, hi = carry
            mid = (lo + hi) // 2
            c = jnp.sum((bits >= mid).astype(jnp.int32))
            big = c >= K
            return (jnp.where(big, mid, lo), jnp.where(big, hi, mid))

        t, _ = jax.lax.fori_loop(
            0, 30, vbody, (jnp.int32(0), jnp.int32(1 << 30))
        )

        gt = bits > t
        eq = bits == t
        n_gt = jnp.sum(gt.astype(jnp.int32))
        need = K - n_gt
        idx = jax.lax.broadcasted_iota(jnp.int32, (1, F), 1)

        def ibody(_, carry):
            lo, hi = carry
            mid = (lo + hi + 1) // 2
            c = jnp.sum((eq & (idx < mid)).astype(jnp.int32))
            ok = c <= need
            return (jnp.where(ok, mid, lo), jnp.where(ok, hi, mid - 1))

        cut, _ = jax.lax.fori_loop(
            0, 14, ibody, (jnp.int32(0), jnp.int32(F))
        )

        sel = gt | (eq & (idx < cut))
        maxv = jnp.max(m)
        e = jnp.where(sel, jnp.exp(m - maxv), 0.0)
        s = jnp.sum(e)
        prob_ref[...] = e * (1.0 / s)

    o_ref[...] = x_ref[...] * prob_ref[...]


def kernel(inputs, mask):
    b = inputs.shape[0]
    return pl.pallas_call(
        _fused,
        grid=(b // TILE,),
        in_specs=[
            pl.BlockSpec((1, F), lambda i: (0, 0)),
            pl.BlockSpec((TILE, F), lambda i: (i, 0)),
        ],
        out_specs=pl.BlockSpec((TILE, F), lambda i: (i, 0)),
        out_shape=jax.ShapeDtypeStruct(inputs.shape, inputs.dtype),
        scratch_shapes=[pltpu.VMEM((1, F), jnp.float32)],
    )(mask, inputs)
